# baseline (speedup 1.0000x reference)
import functools

import jax
import jax.numpy as jnp
from jax import lax
from jax.experimental import pallas as pl
from jax.experimental.pallas import tpu as pltpu
from jax.experimental.pallas import tpu_sc as plsc

DI, DR, P, G = 20000, 20000, 5000, 5000
F, K1 = 256, 128
NRG = 3 * DI + 3 * DR + 2 * P + 2 * G

CH = 512
NT = 16
NSC = 2
SENT = 3_000_000
TRASH = 128

CNT_GCN_SIZES = [DI, DR, DI + DR, DI + P, DR + P, DI + G, DR + G]
CNT_GCN_OFFS = [0, 20000, 40000, 80000, 105000, 130000, 155000]
CNT_RG_OFF = 180000
CNT_TOTAL = CNT_RG_OFF + 18 * NRG
CNT_HALF = 1350016
EP_R = 102400


def _ceil_to(x, m):
    return (x + m - 1) // m * m


def _sc_mesh():
    return plsc.VectorSubcoreMesh(core_axis_name="c", subcore_axis_name="s")


def _zero_stripe(zbuf, acc, sid, rows_pt, zrows):
    for off in range(0, rows_pt, zrows):
        sz = min(zrows, rows_pt - off)
        pltpu.sync_copy(zbuf.at[pl.ds(0, sz)], acc.at[pl.ds(sid * rows_pt + off, sz)])


def _counts_body(gdst, out, dbuf, sidx, ones_v, zbuf, acc):
    cid = lax.axis_index("c")
    sid = lax.axis_index("s")
    lo = cid * CNT_HALF
    accn = CNT_HALF + TRASH
    stripe = accn // NT

    def init16(i, _):
        ones_v[pl.ds(i * 16, 16)] = jnp.full((16,), 1.0, jnp.float32)
        return 0
    lax.fori_loop(0, CH // 16, init16, 0)

    def z16(i, _):
        zbuf[pl.ds(i * 16, 16)] = jnp.zeros((16,), jnp.float32)
        return 0
    lax.fori_loop(0, 8192 // 16, z16, 0)
    for off in range(0, stripe, 8192):
        sz = min(8192, stripe - off)
        pltpu.sync_copy(zbuf.at[pl.ds(0, sz)], acc.at[pl.ds(sid * stripe + off, sz)])
    plsc.subcore_barrier()

    tot = gdst.shape[0]
    per_tile = tot // NT
    nch = per_tile // CH
    lane = lax.iota(jnp.int32, 16)

    def chunk(ch_i, _):
        base = sid * per_tile + ch_i * CH
        pltpu.sync_copy(gdst.at[pl.ds(base, CH)], dbuf)

        def xf(k, _):
            v = dbuf[pl.ds(k * 16, 16)]
            loc = v - lo
            ok = jnp.logical_and(loc >= 0, loc < CNT_HALF)
            tr = CNT_HALF + ((k * 16 + lane) & (TRASH - 1))
            sidx[pl.ds(k * 16, 16)] = jnp.where(ok, loc, tr)
            return 0
        lax.fori_loop(0, CH // 16, xf, 0)
        pltpu.sync_copy(ones_v, acc.at[sidx], add=True)
        return 0
    lax.fori_loop(0, nch, chunk, 0)
    plsc.subcore_barrier()

    wstripe = CNT_HALF // NT
    for off in range(0, wstripe, 8192):
        sz = min(8192, wstripe - off)
        pltpu.sync_copy(acc.at[pl.ds(sid * wstripe + off, sz)], zbuf.at[pl.ds(0, sz)])
        pltpu.sync_copy(zbuf.at[pl.ds(0, sz)],
                        out.at[pl.ds(cid * CNT_HALF + sid * wstripe + off, sz)])


def _counts_call(gdst_all):
    tot = gdst_all.shape[0]
    k = pl.kernel(
        _counts_body,
        out_type=jax.ShapeDtypeStruct((2 * CNT_HALF,), jnp.float32),
        mesh=_sc_mesh(),
        scratch_types=[
            pltpu.VMEM((CH,), jnp.int32),
            pltpu.VMEM((CH,), jnp.int32),
            pltpu.VMEM((CH,), jnp.float32),
            pltpu.VMEM((8192,), jnp.float32),
            pltpu.VMEM_SHARED((CNT_HALF + TRASH,), jnp.float32),
        ],
        name="sc_counts",
        compiler_params=pltpu.CompilerParams(use_tc_tiling_on_sc=False),
    )
    return k(gdst_all)


def _gcn_seg_body(src, dst, x8, out, gidx0, sidx0, gbuf0, gidx1, sidx1, gbuf1,
                  tbuf, zbuf, wbuf, acc, sem0, sem1, *, ept, nd_pad, chunk):
    cid = lax.axis_index("c")
    sid = lax.axis_index("s")
    nch = ept // chunk
    accn = nd_pad + TRASH
    rows_pt = accn // NT
    lane = lax.iota(jnp.int32, 16)
    ebase = sid * ept
    bufs = ((gidx0, sidx0, gbuf0, sem0), (gidx1, sidx1, gbuf1, sem1))

    def zb16(i, _):
        zbuf[i, pl.ds(0, 16)] = jnp.zeros((16,), jnp.float32)
        return 0
    lax.fori_loop(0, 1024, zb16, 0)

    for cg in range(4):
        c = cid * 4 + cg

        def prep_fire(ch_i, b):
            gx, sx, gb, sm = bufs[b]
            base = ebase + ch_i * chunk
            pltpu.sync_copy(src.at[pl.ds(base, chunk)], tbuf)

            def xf(k, _):
                gx[pl.ds(k * 16, 16)] = tbuf[pl.ds(k * 16, 16)] * 8 + c
                return 0
            lax.fori_loop(0, chunk // 16, xf, 0)
            pltpu.sync_copy(dst.at[pl.ds(base, chunk)], tbuf)

            def xf2(k, _):
                d = tbuf[pl.ds(k * 16, 16)]
                ok = d < nd_pad
                tr = nd_pad + ((k * 16 + lane) & (TRASH - 1))
                sx[pl.ds(k * 16, 16)] = jnp.where(ok, d, tr)
                return 0
            lax.fori_loop(0, chunk // 16, xf2, 0)
            pltpu.async_copy(x8.at[gx], gb, sm)

        def wait_scatter(b):
            gx, sx, gb, sm = bufs[b]
            pltpu.make_async_copy(x8.at[gx], gb, sm).wait()
            pltpu.sync_copy(gb, acc.at[sx], add=True)

        _zero_stripe(zbuf, acc, sid, rows_pt, 1024)
        plsc.subcore_barrier()

        prep_fire(0, 0)

        def pair(j, _):
            prep_fire(2 * j + 1, 1)
            wait_scatter(0)

            @pl.when(2 * j + 2 < nch)
            def _():
                prep_fire(2 * j + 2, 0)
            wait_scatter(1)
            return 0
        lax.fori_loop(0, nch // 2, pair, 0)
        if nch % 2 == 1:
            wait_scatter(0)
        plsc.subcore_barrier()

        wrows = nd_pad // NT
        for off in range(0, wrows, 1024):
            sz = min(1024, wrows - off)
            pltpu.sync_copy(acc.at[pl.ds(sid * wrows + off, sz)], wbuf.at[pl.ds(0, sz)])
            pltpu.sync_copy(wbuf.at[pl.ds(0, sz)],
                            out.at[pl.ds(sid * wrows + off, sz), pl.ds(c * 16, 16)])
        plsc.subcore_barrier()


GCH = 1024


def _gcn_seg_call(src, dst, hs, nd_pad):
    ns = hs.shape[0]
    e_pad = src.shape[0]
    ept = e_pad // NT
    x8 = hs.reshape(ns * 8, 16)
    k = pl.kernel(
        functools.partial(_gcn_seg_body, ept=ept, nd_pad=nd_pad, chunk=GCH),
        out_type=jax.ShapeDtypeStruct((nd_pad, 128), jnp.float32),
        mesh=_sc_mesh(),
        scratch_types=[
            pltpu.VMEM((GCH,), jnp.int32),
            pltpu.VMEM((GCH,), jnp.int32),
            pltpu.VMEM((GCH, 16), jnp.float32),
            pltpu.VMEM((GCH,), jnp.int32),
            pltpu.VMEM((GCH,), jnp.int32),
            pltpu.VMEM((GCH, 16), jnp.float32),
            pltpu.VMEM((GCH,), jnp.int32),
            pltpu.VMEM((1024, 16), jnp.float32),
            pltpu.VMEM((1024, 16), jnp.float32),
            pltpu.VMEM_SHARED((nd_pad + TRASH, 16), jnp.float32),
            pltpu.SemaphoreType.DMA,
            pltpu.SemaphoreType.DMA,
        ],
        name="sc_gcn_segsum",
        compiler_params=pltpu.CompilerParams(use_tc_tiling_on_sc=False),
    )
    return k(src, dst, x8)


RG_HALF = NRG // 2


RCH = 256


def _rg_seg_body(src, dst, x8, out, gb_c, ds_c, gidx0, sidx0, gbuf0, gidx1,
                 sidx1, gbuf1, tbuf, zbuf, wbuf, acc, sem0, sem1):
    cid = lax.axis_index("c")
    sid = lax.axis_index("s")
    ept = EP_R // NT
    nch = ept // RCH
    accn = NRG + TRASH
    rows_pt = accn // NT
    lane = lax.iota(jnp.int32, 16)
    bufs = ((gidx0, sidx0, gbuf0, sem0), (gidx1, sidx1, gbuf1, sem1))

    def zb2(i, _):
        zbuf[pl.ds(2 * i, 2), pl.ds(0, 16)] = jnp.zeros((2, 16), jnp.bfloat16)
        return 0
    lax.fori_loop(0, 512, zb2, 0)

    def per_dir(r, _):
        dbase = r * EP_R + sid * ept

        def ld(ch_i, _):
            pltpu.sync_copy(src.at[pl.ds(dbase + ch_i * RCH, RCH)], tbuf)

            def xf(k, _):
                gb_c[pl.ds(ch_i * RCH + k * 16, 16)] = tbuf[pl.ds(k * 16, 16)] * 8
                return 0
            lax.fori_loop(0, RCH // 16, xf, 0)
            pltpu.sync_copy(dst.at[pl.ds(dbase + ch_i * RCH, RCH)], tbuf)

            def xf2(k, _):
                ds_c[pl.ds(ch_i * RCH + k * 16, 16)] = tbuf[pl.ds(k * 16, 16)]
                return 0
            lax.fori_loop(0, RCH // 16, xf2, 0)
            return 0
        lax.fori_loop(0, nch, ld, 0)

        def cg_body(cg, _):
            c = cid * 4 + cg

            def prep_fire(ch_i, b):
                gx, sx, gb, sm = bufs[b]

                def xf(k, _):
                    gx[pl.ds(k * 16, 16)] = gb_c[pl.ds(ch_i * RCH + k * 16, 16)] + c
                    d = ds_c[pl.ds(ch_i * RCH + k * 16, 16)]
                    ok = d < NRG
                    tr = NRG + ((k * 16 + lane) & (TRASH - 1))
                    sx[pl.ds(k * 16, 16)] = jnp.where(ok, d, tr)
                    return 0
                lax.fori_loop(0, RCH // 16, xf, 0)
                pltpu.async_copy(x8.at[gx], gb, sm)

            def wait_scatter(b):
                gx, sx, gb, sm = bufs[b]
                pltpu.make_async_copy(x8.at[gx], gb, sm).wait()
                pltpu.sync_copy(gb, acc.at[sx], add=True)

            _zero_stripe(zbuf, acc, sid, rows_pt, 1024)
            plsc.subcore_barrier()

            prep_fire(0, 0)

            def pair(j, _):
                prep_fire(2 * j + 1, 1)
                wait_scatter(0)

                @pl.when(2 * j + 2 < nch)
                def _():
                    prep_fire(2 * j + 2, 0)
                wait_scatter(1)
                return 0
            lax.fori_loop(0, nch // 2, pair, 0)
            if (EP_R // NT // RCH) % 2 == 1:
                wait_scatter(0)
            plsc.subcore_barrier()

            wrows = NRG // NT
            orow = r * NRG + sid * wrows
            for off in range(0, wrows, 1024):
                sz = min(1024, wrows - off)
                pltpu.sync_copy(acc.at[pl.ds(sid * wrows + off, sz)],
                                wbuf.at[pl.ds(0, sz)])
                pltpu.sync_copy(wbuf.at[pl.ds(0, sz)],
                                out.at[pl.ds(orow + off, sz), pl.ds(c * 16, 16)])
            plsc.subcore_barrier()
            return 0
        lax.fori_loop(0, 4, cg_body, 0)
        return 0
    lax.fori_loop(0, 18, per_dir, 0)


def _rg_seg_call(src_all, dst_all, xr_bf):
    ept = EP_R // NT
    x8 = xr_bf.reshape(NRG * 8, 16)
    k = pl.kernel(
        _rg_seg_body,
        out_type=jax.ShapeDtypeStruct((18 * NRG, 128), jnp.bfloat16),
        mesh=_sc_mesh(),
        scratch_types=[
            pltpu.VMEM((ept,), jnp.int32),
            pltpu.VMEM((ept,), jnp.int32),
            pltpu.VMEM((RCH,), jnp.int32),
            pltpu.VMEM((RCH,), jnp.int32),
            pltpu.VMEM((RCH, 16), jnp.bfloat16),
            pltpu.VMEM((RCH,), jnp.int32),
            pltpu.VMEM((RCH,), jnp.int32),
            pltpu.VMEM((RCH, 16), jnp.bfloat16),
            pltpu.VMEM((RCH,), jnp.int32),
            pltpu.VMEM((1024, 16), jnp.bfloat16),
            pltpu.VMEM((1024, 16), jnp.bfloat16),
            pltpu.VMEM_SHARED((NRG + TRASH, 16), jnp.bfloat16),
            pltpu.SemaphoreType.DMA,
            pltpu.SemaphoreType.DMA,
        ],
        name="sc_rgcn_segsum",
        compiler_params=pltpu.CompilerParams(use_tc_tiling_on_sc=False),
    )
    return k(src_all, dst_all, x8)


BM = 1000


def _mm_body(x, w, cnt, o, *, scale, relu_):
    acc = jnp.dot(x[...], w[...], preferred_element_type=jnp.float32)
    if scale:
        acc = acc * lax.rsqrt(1.0 + cnt[...])
    else:
        acc = acc + cnt[...]
    if relu_:
        acc = jnp.maximum(acc, 0.0)
    o[...] = acc


def _mm(x, w, cnt2, scale, relu_):
    n, kin = x.shape
    ko = w.shape[1]
    cw = cnt2.shape[1]
    return pl.pallas_call(
        functools.partial(_mm_body, scale=scale, relu_=relu_),
        grid=(n // BM,),
        in_specs=[
            pl.BlockSpec((BM, kin), lambda i: (i, 0)),
            pl.BlockSpec((kin, ko), lambda i: (0, 0)),
            pl.BlockSpec((BM, cw) if scale else (1, cw), (lambda i: (i, 0)) if scale else (lambda i: (0, 0))),
        ],
        out_specs=pl.BlockSpec((BM, ko), lambda i: (i, 0)),
        out_shape=jax.ShapeDtypeStruct((n, ko), jnp.float32),
    )(x, w, cnt2)


def _mm_scale(x, w, cnt):
    return _mm(x, w, cnt.reshape(-1, 1), True, False)


def _lin_relu(x, w, b):
    return _mm(x, w, b.reshape(1, -1), False, True)


def _finish_body(agg, hs, cnt, b, o):
    dis = lax.rsqrt(1.0 + cnt[...])
    o[...] = jnp.maximum((agg[...] + hs[...]) * dis + b[...], 0.0)


def _finish(agg, hs, cnt, b):
    n = hs.shape[0]
    return pl.pallas_call(
        _finish_body,
        grid=(n // BM,),
        in_specs=[
            pl.BlockSpec((BM, 128), lambda i: (i, 0)),
            pl.BlockSpec((BM, 128), lambda i: (i, 0)),
            pl.BlockSpec((BM, 1), lambda i: (i, 0)),
            pl.BlockSpec((1, 128), lambda i: (0, 0)),
        ],
        out_specs=pl.BlockSpec((BM, 128), lambda i: (i, 0)),
        out_shape=jax.ShapeDtypeStruct((n, 128), jnp.float32),
    )(agg, hs, cnt.reshape(-1, 1), b.reshape(1, -1))


CBM = 560


def _combine_body(xr, a, cnt, wroot, wrel, b, o):
    acc = jnp.dot(xr[...], wroot[...], preferred_element_type=jnp.float32) + b[...]
    for r in range(18):
        invc = 1.0 / jnp.maximum(cnt[r], 1.0)
        acc = acc + jnp.dot(a[r].astype(jnp.float32) * invc, wrel[r],
                            preferred_element_type=jnp.float32)
    o[...] = jnp.maximum(acc, 0.0)


def _combine(xr, a3, cnt_rg, wroot, wrel, b):
    n = xr.shape[0]
    return pl.pallas_call(
        _combine_body,
        grid=(n // CBM,),
        in_specs=[
            pl.BlockSpec((CBM, 128), lambda i: (i, 0)),
            pl.BlockSpec((18, CBM, 128), lambda i: (0, i, 0)),
            pl.BlockSpec((18, CBM, 1), lambda i: (0, i, 0)),
            pl.BlockSpec((128, 128), lambda i: (0, 0)),
            pl.BlockSpec((18, 128, 128), lambda i: (0, 0, 0)),
            pl.BlockSpec((1, 128), lambda i: (0, 0)),
        ],
        out_specs=pl.BlockSpec((CBM, 128), lambda i: (i, 0)),
        out_shape=jax.ShapeDtypeStruct((n, 128), jnp.float32),
    )(xr, a3, cnt_rg.reshape(18, NRG, 1), wroot, wrel, b.reshape(1, -1))


def _pad_e(src, dst, e_pad, ns):
    e = src.shape[0]
    padn = e_pad - e
    if padn == 0:
        return src, dst
    psrc = (jnp.arange(padn, dtype=jnp.int32) % ns)
    pdst = jnp.full((padn,), SENT, jnp.int32)
    return jnp.concatenate([src, psrc]), jnp.concatenate([dst, pdst])


def _both_dirs(ei):
    return jnp.concatenate([ei[0], ei[1]]), jnp.concatenate([ei[1], ei[0]])


def kernel(di_data, dr_data, p_data, g_data, W_lin_di, b_lin_di, W_lin_dr, b_lin_dr,
           W_lin_p, b_lin_p, W_lin_g, b_lin_g, W_didi, b_didi, W_drdr, b_drdr,
           W_drdi, b_drdi, W_dip, b_dip, W_drp, b_drp, W_dig, b_dig, W_drg, b_drg,
           W_rel, W_root, b_rgcn, di_edge_index, dr_edge_index, didr_edge_index,
           dip_edge_gcn, drp_edge_gcn, dig_edge_gcn, drg_edge_gcn, e_di1di3,
           e_di1dr1, e_di1dr2, e_di2dr1, e_dr1dr3, e_p3di1, e_p2dr1, e_g3di1,
           e_g2dr1):
    jobs = {}
    s, d = di_edge_index[0], di_edge_index[1]
    jobs['di'] = _pad_e(s, d, _ceil_to(s.shape[0], NT * GCH), DI) + (DI, DI)
    s, d = dr_edge_index[0], dr_edge_index[1]
    jobs['dr'] = _pad_e(s, d, _ceil_to(s.shape[0], NT * GCH), DR) + (DR, DR)
    for nm, ei, nn in (('didr', didr_edge_index, DI + DR),
                       ('dip', dip_edge_gcn, DI + P),
                       ('drp', drp_edge_gcn, DR + P),
                       ('dig', dig_edge_gcn, DI + G),
                       ('drg', drg_edge_gcn, DR + G)):
        s, d = _both_dirs(ei)
        jobs[nm] = _pad_e(s, d, _ceil_to(s.shape[0], NT * GCH), nn) + (nn, _ceil_to(nn, NT))

    rel_lists = [e_di1di3, e_di1dr1, e_di1dr2, e_p3di1, e_di2dr1,
                 e_dr1dr3, e_p2dr1, e_g3di1, e_g2dr1]
    rsrcs, rdsts = [], []
    for ee in rel_lists:
        for (s, d) in ((ee[0], ee[1]), (ee[1], ee[0])):
            ps, pd = _pad_e(s, d, EP_R, NRG)
            rsrcs.append(ps)
            rdsts.append(pd)
    rsrc_all = jnp.concatenate(rsrcs)
    rdst_all = jnp.concatenate(rdsts)

    order = ['di', 'dr', 'didr', 'dip', 'drp', 'dig', 'drg']
    gdsts = [jobs[nm][1] + off for nm, off in zip(order, CNT_GCN_OFFS)]
    dir_off = jnp.repeat(
        CNT_RG_OFF + NRG * jnp.arange(18, dtype=jnp.int32), EP_R)
    gdsts.append(rdst_all + dir_off)
    gdst_all = jnp.concatenate(gdsts)

    counts = _counts_call(gdst_all)
    cnt = {nm: lax.dynamic_slice(counts, (off,), (sz,))
           for nm, off, sz in zip(order, CNT_GCN_OFFS, CNT_GCN_SIZES)}
    cnt_rg = counts[CNT_RG_OFF:CNT_RG_OFF + 18 * NRG].reshape(18, NRG)

    hs_di = _mm_scale(di_data, W_didi, cnt['di'])
    agg_di = _gcn_seg_call(*jobs['di'][:2], hs_di, jobs['di'][3])
    di1 = _finish(agg_di[:DI], hs_di, cnt['di'], b_didi)

    hs_dr = _mm_scale(dr_data, W_drdr, cnt['dr'])
    agg_dr = _gcn_seg_call(*jobs['dr'][:2], hs_dr, jobs['dr'][3])
    dr1 = _finish(agg_dr[:DR], hs_dr, cnt['dr'], b_drdr)

    di2 = _lin_relu(di_data, W_lin_di, b_lin_di)
    dr2 = _lin_relu(dr_data, W_lin_dr, b_lin_dr)
    p1 = _lin_relu(p_data, W_lin_p, b_lin_p)
    g1 = _lin_relu(g_data, W_lin_g, b_lin_g)

    def layer2(nm, xa, xb, w, b):
        src, dst, nn, nd_pad = jobs[nm]
        c = cnt[nm]
        na = xa.shape[0]
        hs = jnp.concatenate([_mm_scale(xa, w, c[:na]), _mm_scale(xb, w, c[na:])])
        if hs.shape[0] < nd_pad:
            hs = jnp.pad(hs, ((0, nd_pad - hs.shape[0]), (0, 0)))
        agg = _gcn_seg_call(src, dst, hs, nd_pad)
        return _finish(agg[:nn], hs[:nn], c, b)

    h = layer2('didr', di2, dr2, W_drdi, b_drdi)
    di3, dr3 = h[:DI], h[DI:]
    p3 = layer2('dip', di2, p1, W_dip, b_dip)[DI:]
    p2 = layer2('drp', dr2, p1, W_drp, b_drp)[DR:]
    g3 = layer2('dig', di2, g1, W_dig, b_dig)[DI:]
    g2 = layer2('drg', dr2, g1, W_drg, b_drg)[DR:]

    xr = jnp.concatenate([di2, di1, di3, dr2, dr1, dr3, p3, p2, g3, g2])
    a = _rg_seg_call(rsrc_all, rdst_all, xr.astype(jnp.bfloat16))
    return _combine(xr, a.reshape(18, NRG, 128), cnt_rg, W_root, W_rel, b_rgcn)


# speedup vs baseline: 1.0996x; 1.0996x over previous
import functools

import jax
import jax.numpy as jnp
from jax import lax
from jax.experimental import pallas as pl
from jax.experimental.pallas import tpu as pltpu
from jax.experimental.pallas import tpu_sc as plsc

DI, DR, P, G = 20000, 20000, 5000, 5000
F, K1 = 256, 128
NRG = 3 * DI + 3 * DR + 2 * P + 2 * G

CH = 512
NT = 16
NSC = 2
SENT = 3_000_000
TRASH = 128

CNT_GCN_SIZES = [DI, DR, DI + DR, DI + P, DR + P, DI + G, DR + G]
CNT_GCN_OFFS = [0, 20000, 40000, 80000, 105000, 130000, 155000]
CNT_RG_OFF = 180000
CNT_TOTAL = CNT_RG_OFF + 18 * NRG
CNT_HALF = 1350016
EP_R = 102400


def _ceil_to(x, m):
    return (x + m - 1) // m * m


def _sc_mesh():
    return plsc.VectorSubcoreMesh(core_axis_name="c", subcore_axis_name="s")


def _zero_stripe(zbuf, acc, sid, rows_pt, zrows):
    for off in range(0, rows_pt, zrows):
        sz = min(zrows, rows_pt - off)
        pltpu.sync_copy(zbuf.at[pl.ds(0, sz)], acc.at[pl.ds(sid * rows_pt + off, sz)])


def _counts_body(gdst, out, dbuf, sidx, ones_v, zbuf, acc):
    cid = lax.axis_index("c")
    sid = lax.axis_index("s")
    lo = cid * CNT_HALF
    accn = CNT_HALF + TRASH
    stripe = accn // NT

    def init16(i, _):
        ones_v[pl.ds(i * 16, 16)] = jnp.full((16,), 1.0, jnp.float32)
        return 0
    lax.fori_loop(0, CH // 16, init16, 0)

    def z16(i, _):
        zbuf[pl.ds(i * 16, 16)] = jnp.zeros((16,), jnp.float32)
        return 0
    lax.fori_loop(0, 8192 // 16, z16, 0)
    for off in range(0, stripe, 8192):
        sz = min(8192, stripe - off)
        pltpu.sync_copy(zbuf.at[pl.ds(0, sz)], acc.at[pl.ds(sid * stripe + off, sz)])
    plsc.subcore_barrier()

    tot = gdst.shape[0]
    per_tile = tot // NT
    nch = per_tile // CH
    lane = lax.iota(jnp.int32, 16)

    def chunk(ch_i, _):
        base = sid * per_tile + ch_i * CH
        pltpu.sync_copy(gdst.at[pl.ds(base, CH)], dbuf)

        def xf(k, _):
            v = dbuf[pl.ds(k * 16, 16)]
            loc = v - lo
            ok = jnp.logical_and(loc >= 0, loc < CNT_HALF)
            tr = CNT_HALF + ((k * 16 + lane) & (TRASH - 1))
            sidx[pl.ds(k * 16, 16)] = jnp.where(ok, loc, tr)
            return 0
        lax.fori_loop(0, CH // 16, xf, 0)
        pltpu.sync_copy(ones_v, acc.at[sidx], add=True)
        return 0
    lax.fori_loop(0, nch, chunk, 0)
    plsc.subcore_barrier()

    wstripe = CNT_HALF // NT
    for off in range(0, wstripe, 8192):
        sz = min(8192, wstripe - off)
        pltpu.sync_copy(acc.at[pl.ds(sid * wstripe + off, sz)], zbuf.at[pl.ds(0, sz)])
        pltpu.sync_copy(zbuf.at[pl.ds(0, sz)],
                        out.at[pl.ds(cid * CNT_HALF + sid * wstripe + off, sz)])


def _counts_call(gdst_all):
    tot = gdst_all.shape[0]
    k = pl.kernel(
        _counts_body,
        out_type=jax.ShapeDtypeStruct((2 * CNT_HALF,), jnp.float32),
        mesh=_sc_mesh(),
        scratch_types=[
            pltpu.VMEM((CH,), jnp.int32),
            pltpu.VMEM((CH,), jnp.int32),
            pltpu.VMEM((CH,), jnp.float32),
            pltpu.VMEM((8192,), jnp.float32),
            pltpu.VMEM_SHARED((CNT_HALF + TRASH,), jnp.float32),
        ],
        name="sc_counts",
        compiler_params=pltpu.CompilerParams(use_tc_tiling_on_sc=False),
    )
    return k(gdst_all)


def _gcn_seg_body(src, dst, x8, out, gidx0, sidx0, gbuf0, gidx1, sidx1, gbuf1,
                  tbuf, zbuf, wbuf, acc, sem0, sem1, *, ept, nd_pad, chunk):
    cid = lax.axis_index("c")
    sid = lax.axis_index("s")
    nch = ept // chunk
    accn = nd_pad + TRASH
    rows_pt = accn // NT
    lane = lax.iota(jnp.int32, 16)
    ebase = sid * ept
    bufs = ((gidx0, sidx0, gbuf0, sem0), (gidx1, sidx1, gbuf1, sem1))

    def zb16(i, _):
        zbuf[i, pl.ds(0, 16)] = jnp.zeros((16,), jnp.float32)
        return 0
    lax.fori_loop(0, 1024, zb16, 0)

    for cg in range(4):
        c = cid * 4 + cg

        def prep_fire(ch_i, b):
            gx, sx, gb, sm = bufs[b]
            base = ebase + ch_i * chunk
            pltpu.sync_copy(src.at[pl.ds(base, chunk)], tbuf)

            def xf(k, _):
                gx[pl.ds(k * 16, 16)] = tbuf[pl.ds(k * 16, 16)] * 8 + c
                return 0
            lax.fori_loop(0, chunk // 16, xf, 0)
            pltpu.sync_copy(dst.at[pl.ds(base, chunk)], tbuf)

            def xf2(k, _):
                d = tbuf[pl.ds(k * 16, 16)]
                ok = d < nd_pad
                tr = nd_pad + ((k * 16 + lane) & (TRASH - 1))
                sx[pl.ds(k * 16, 16)] = jnp.where(ok, d, tr)
                return 0
            lax.fori_loop(0, chunk // 16, xf2, 0)
            pltpu.async_copy(x8.at[gx], gb, sm)

        def wait_scatter(b):
            gx, sx, gb, sm = bufs[b]
            pltpu.make_async_copy(x8.at[gx], gb, sm).wait()
            pltpu.sync_copy(gb, acc.at[sx], add=True)

        _zero_stripe(zbuf, acc, sid, rows_pt, 1024)
        plsc.subcore_barrier()

        prep_fire(0, 0)

        def pair(j, _):
            prep_fire(2 * j + 1, 1)
            wait_scatter(0)

            @pl.when(2 * j + 2 < nch)
            def _():
                prep_fire(2 * j + 2, 0)
            wait_scatter(1)
            return 0
        lax.fori_loop(0, nch // 2, pair, 0)
        if nch % 2 == 1:
            wait_scatter(0)
        plsc.subcore_barrier()

        wrows = nd_pad // NT
        for off in range(0, wrows, 1024):
            sz = min(1024, wrows - off)
            pltpu.sync_copy(acc.at[pl.ds(sid * wrows + off, sz)], wbuf.at[pl.ds(0, sz)])
            pltpu.sync_copy(wbuf.at[pl.ds(0, sz)],
                            out.at[pl.ds(sid * wrows + off, sz), pl.ds(c * 16, 16)])
        plsc.subcore_barrier()


GCH = 1024


def _gcn_seg_call(src, dst, hs, nd_pad):
    ns = hs.shape[0]
    e_pad = src.shape[0]
    ept = e_pad // NT
    x8 = hs.reshape(ns * 8, 16)
    k = pl.kernel(
        functools.partial(_gcn_seg_body, ept=ept, nd_pad=nd_pad, chunk=GCH),
        out_type=jax.ShapeDtypeStruct((nd_pad, 128), jnp.float32),
        mesh=_sc_mesh(),
        scratch_types=[
            pltpu.VMEM((GCH,), jnp.int32),
            pltpu.VMEM((GCH,), jnp.int32),
            pltpu.VMEM((GCH, 16), jnp.float32),
            pltpu.VMEM((GCH,), jnp.int32),
            pltpu.VMEM((GCH,), jnp.int32),
            pltpu.VMEM((GCH, 16), jnp.float32),
            pltpu.VMEM((GCH,), jnp.int32),
            pltpu.VMEM((1024, 16), jnp.float32),
            pltpu.VMEM((1024, 16), jnp.float32),
            pltpu.VMEM_SHARED((nd_pad + TRASH, 16), jnp.float32),
            pltpu.SemaphoreType.DMA,
            pltpu.SemaphoreType.DMA,
        ],
        name="sc_gcn_segsum",
        compiler_params=pltpu.CompilerParams(use_tc_tiling_on_sc=False),
    )
    return k(src, dst, x8)


RG_HALF = NRG // 2


RCH = 256


def _rg_seg_body(src, dst, x8, out, gb_c, ds_c, gidx0, sidx0, gbuf0, gidx1,
                 sidx1, gbuf1, tbuf, zbuf, wbuf, acc, sem0, sem1):
    cid = lax.axis_index("c")
    sid = lax.axis_index("s")
    ept = EP_R // NT
    nch = ept // RCH
    lo = cid * RG_HALF
    accn = RG_HALF + TRASH
    rows_pt = accn // NT
    lane = lax.iota(jnp.int32, 16)
    bufs = ((gidx0, sidx0, gbuf0, sem0), (gidx1, sidx1, gbuf1, sem1))

    def zb16(i, _):
        zbuf[i, pl.ds(0, 16)] = jnp.zeros((16,), jnp.float32)
        return 0
    lax.fori_loop(0, 1024, zb16, 0)

    def per_dir(r, _):
        dbase = r * EP_R + sid * ept

        def ld(ch_i, _):
            pltpu.sync_copy(src.at[pl.ds(dbase + ch_i * RCH, RCH)], tbuf)

            def xf(k, _):
                gb_c[pl.ds(ch_i * RCH + k * 16, 16)] = tbuf[pl.ds(k * 16, 16)] * 8
                return 0
            lax.fori_loop(0, RCH // 16, xf, 0)
            pltpu.sync_copy(dst.at[pl.ds(dbase + ch_i * RCH, RCH)], tbuf)

            def xf2(k, _):
                ds_c[pl.ds(ch_i * RCH + k * 16, 16)] = tbuf[pl.ds(k * 16, 16)]
                return 0
            lax.fori_loop(0, RCH // 16, xf2, 0)
            return 0
        lax.fori_loop(0, nch, ld, 0)

        def cg_body(cg, _):
            def prep_fire(ch_i, b):
                gx, sx, gb, sm = bufs[b]

                def xf(k, _):
                    gx[pl.ds(k * 16, 16)] = gb_c[pl.ds(ch_i * RCH + k * 16, 16)] + cg
                    d = ds_c[pl.ds(ch_i * RCH + k * 16, 16)] - lo
                    ok = jnp.logical_and(d >= 0, d < RG_HALF)
                    tr = RG_HALF + ((k * 16 + lane) & (TRASH - 1))
                    sx[pl.ds(k * 16, 16)] = jnp.where(ok, d, tr)
                    return 0
                lax.fori_loop(0, RCH // 16, xf, 0)
                pltpu.async_copy(x8.at[gx], gb, sm)

            def wait_scatter(b):
                gx, sx, gb, sm = bufs[b]
                pltpu.make_async_copy(x8.at[gx], gb, sm).wait()
                pltpu.sync_copy(gb, acc.at[sx], add=True)

            _zero_stripe(zbuf, acc, sid, rows_pt, 1024)
            plsc.subcore_barrier()

            prep_fire(0, 0)

            def pair(j, _):
                prep_fire(2 * j + 1, 1)
                wait_scatter(0)

                @pl.when(2 * j + 2 < nch)
                def _():
                    prep_fire(2 * j + 2, 0)
                wait_scatter(1)
                return 0
            lax.fori_loop(0, nch // 2, pair, 0)
            if (EP_R // NT // RCH) % 2 == 1:
                wait_scatter(0)
            plsc.subcore_barrier()

            wrows = RG_HALF // NT
            orow = r * NRG + lo + sid * wrows
            for off in range(0, wrows, 1024):
                sz = min(1024, wrows - off)
                pltpu.sync_copy(acc.at[pl.ds(sid * wrows + off, sz)],
                                wbuf.at[pl.ds(0, sz)])
                pltpu.sync_copy(wbuf.at[pl.ds(0, sz)],
                                out.at[pl.ds(orow + off, sz), pl.ds(cg * 16, 16)])
            plsc.subcore_barrier()
            return 0
        lax.fori_loop(0, 8, cg_body, 0)
        return 0
    lax.fori_loop(0, 18, per_dir, 0)


def _rg_seg_call(src_all, dst_all, xr):
    ept = EP_R // NT
    x8 = xr.reshape(NRG * 8, 16)
    k = pl.kernel(
        _rg_seg_body,
        out_type=jax.ShapeDtypeStruct((18 * NRG, 128), jnp.float32),
        mesh=_sc_mesh(),
        scratch_types=[
            pltpu.VMEM((ept,), jnp.int32),
            pltpu.VMEM((ept,), jnp.int32),
            pltpu.VMEM((RCH,), jnp.int32),
            pltpu.VMEM((RCH,), jnp.int32),
            pltpu.VMEM((RCH, 16), jnp.float32),
            pltpu.VMEM((RCH,), jnp.int32),
            pltpu.VMEM((RCH,), jnp.int32),
            pltpu.VMEM((RCH, 16), jnp.float32),
            pltpu.VMEM((RCH,), jnp.int32),
            pltpu.VMEM((1024, 16), jnp.float32),
            pltpu.VMEM((1024, 16), jnp.float32),
            pltpu.VMEM_SHARED((RG_HALF + TRASH, 16), jnp.float32),
            pltpu.SemaphoreType.DMA,
            pltpu.SemaphoreType.DMA,
        ],
        name="sc_rgcn_segsum",
        compiler_params=pltpu.CompilerParams(use_tc_tiling_on_sc=False),
    )
    return k(src_all, dst_all, x8)


BM = 1000


def _mm_body(x, w, cnt, o, *, scale, relu_):
    acc = jnp.dot(x[...], w[...], preferred_element_type=jnp.float32)
    if scale:
        acc = acc * lax.rsqrt(1.0 + cnt[...])
    else:
        acc = acc + cnt[...]
    if relu_:
        acc = jnp.maximum(acc, 0.0)
    o[...] = acc


def _mm(x, w, cnt2, scale, relu_):
    n, kin = x.shape
    ko = w.shape[1]
    cw = cnt2.shape[1]
    return pl.pallas_call(
        functools.partial(_mm_body, scale=scale, relu_=relu_),
        grid=(n // BM,),
        in_specs=[
            pl.BlockSpec((BM, kin), lambda i: (i, 0)),
            pl.BlockSpec((kin, ko), lambda i: (0, 0)),
            pl.BlockSpec((BM, cw) if scale else (1, cw), (lambda i: (i, 0)) if scale else (lambda i: (0, 0))),
        ],
        out_specs=pl.BlockSpec((BM, ko), lambda i: (i, 0)),
        out_shape=jax.ShapeDtypeStruct((n, ko), jnp.float32),
    )(x, w, cnt2)


def _mm_scale(x, w, cnt):
    return _mm(x, w, cnt.reshape(-1, 1), True, False)


def _lin_relu(x, w, b):
    return _mm(x, w, b.reshape(1, -1), False, True)


def _finish_body(agg, hs, cnt, b, o):
    dis = lax.rsqrt(1.0 + cnt[...])
    o[...] = jnp.maximum((agg[...] + hs[...]) * dis + b[...], 0.0)


def _finish(agg, hs, cnt, b):
    n = hs.shape[0]
    return pl.pallas_call(
        _finish_body,
        grid=(n // BM,),
        in_specs=[
            pl.BlockSpec((BM, 128), lambda i: (i, 0)),
            pl.BlockSpec((BM, 128), lambda i: (i, 0)),
            pl.BlockSpec((BM, 1), lambda i: (i, 0)),
            pl.BlockSpec((1, 128), lambda i: (0, 0)),
        ],
        out_specs=pl.BlockSpec((BM, 128), lambda i: (i, 0)),
        out_shape=jax.ShapeDtypeStruct((n, 128), jnp.float32),
    )(agg, hs, cnt.reshape(-1, 1), b.reshape(1, -1))


CBM = 560


def _combine_body(xr, a, cnt, wroot, wrel, b, o):
    acc = jnp.dot(xr[...], wroot[...], preferred_element_type=jnp.float32) + b[...]
    for r in range(18):
        invc = 1.0 / jnp.maximum(cnt[r], 1.0)
        acc = acc + jnp.dot(a[r].astype(jnp.float32) * invc, wrel[r],
                            preferred_element_type=jnp.float32)
    o[...] = jnp.maximum(acc, 0.0)


def _combine(xr, a3, cnt_rg, wroot, wrel, b):
    n = xr.shape[0]
    return pl.pallas_call(
        _combine_body,
        grid=(n // CBM,),
        in_specs=[
            pl.BlockSpec((CBM, 128), lambda i: (i, 0)),
            pl.BlockSpec((18, CBM, 128), lambda i: (0, i, 0)),
            pl.BlockSpec((18, CBM, 1), lambda i: (0, i, 0)),
            pl.BlockSpec((128, 128), lambda i: (0, 0)),
            pl.BlockSpec((18, 128, 128), lambda i: (0, 0, 0)),
            pl.BlockSpec((1, 128), lambda i: (0, 0)),
        ],
        out_specs=pl.BlockSpec((CBM, 128), lambda i: (i, 0)),
        out_shape=jax.ShapeDtypeStruct((n, 128), jnp.float32),
    )(xr, a3, cnt_rg.reshape(18, NRG, 1), wroot, wrel, b.reshape(1, -1))


def _pad_e(src, dst, e_pad, ns):
    e = src.shape[0]
    padn = e_pad - e
    if padn == 0:
        return src, dst
    psrc = (jnp.arange(padn, dtype=jnp.int32) % ns)
    pdst = jnp.full((padn,), SENT, jnp.int32)
    return jnp.concatenate([src, psrc]), jnp.concatenate([dst, pdst])


def _both_dirs(ei):
    return jnp.concatenate([ei[0], ei[1]]), jnp.concatenate([ei[1], ei[0]])


def kernel(di_data, dr_data, p_data, g_data, W_lin_di, b_lin_di, W_lin_dr, b_lin_dr,
           W_lin_p, b_lin_p, W_lin_g, b_lin_g, W_didi, b_didi, W_drdr, b_drdr,
           W_drdi, b_drdi, W_dip, b_dip, W_drp, b_drp, W_dig, b_dig, W_drg, b_drg,
           W_rel, W_root, b_rgcn, di_edge_index, dr_edge_index, didr_edge_index,
           dip_edge_gcn, drp_edge_gcn, dig_edge_gcn, drg_edge_gcn, e_di1di3,
           e_di1dr1, e_di1dr2, e_di2dr1, e_dr1dr3, e_p3di1, e_p2dr1, e_g3di1,
           e_g2dr1):
    jobs = {}
    s, d = di_edge_index[0], di_edge_index[1]
    jobs['di'] = _pad_e(s, d, _ceil_to(s.shape[0], NT * GCH), DI) + (DI, DI)
    s, d = dr_edge_index[0], dr_edge_index[1]
    jobs['dr'] = _pad_e(s, d, _ceil_to(s.shape[0], NT * GCH), DR) + (DR, DR)
    for nm, ei, nn in (('didr', didr_edge_index, DI + DR),
                       ('dip', dip_edge_gcn, DI + P),
                       ('drp', drp_edge_gcn, DR + P),
                       ('dig', dig_edge_gcn, DI + G),
                       ('drg', drg_edge_gcn, DR + G)):
        s, d = _both_dirs(ei)
        jobs[nm] = _pad_e(s, d, _ceil_to(s.shape[0], NT * GCH), nn) + (nn, _ceil_to(nn, NT))

    rel_lists = [e_di1di3, e_di1dr1, e_di1dr2, e_p3di1, e_di2dr1,
                 e_dr1dr3, e_p2dr1, e_g3di1, e_g2dr1]
    rsrcs, rdsts = [], []
    for ee in rel_lists:
        for (s, d) in ((ee[0], ee[1]), (ee[1], ee[0])):
            ps, pd = _pad_e(s, d, EP_R, NRG)
            rsrcs.append(ps)
            rdsts.append(pd)
    rsrc_all = jnp.concatenate(rsrcs)
    rdst_all = jnp.concatenate(rdsts)

    order = ['di', 'dr', 'didr', 'dip', 'drp', 'dig', 'drg']
    gdsts = [jobs[nm][1] + off for nm, off in zip(order, CNT_GCN_OFFS)]
    dir_off = jnp.repeat(
        CNT_RG_OFF + NRG * jnp.arange(18, dtype=jnp.int32), EP_R)
    gdsts.append(rdst_all + dir_off)
    gdst_all = jnp.concatenate(gdsts)

    counts = _counts_call(gdst_all)
    cnt = {nm: lax.dynamic_slice(counts, (off,), (sz,))
           for nm, off, sz in zip(order, CNT_GCN_OFFS, CNT_GCN_SIZES)}
    cnt_rg = counts[CNT_RG_OFF:CNT_RG_OFF + 18 * NRG].reshape(18, NRG)

    hs_di = _mm_scale(di_data, W_didi, cnt['di'])
    agg_di = _gcn_seg_call(*jobs['di'][:2], hs_di, jobs['di'][3])
    di1 = _finish(agg_di[:DI], hs_di, cnt['di'], b_didi)

    hs_dr = _mm_scale(dr_data, W_drdr, cnt['dr'])
    agg_dr = _gcn_seg_call(*jobs['dr'][:2], hs_dr, jobs['dr'][3])
    dr1 = _finish(agg_dr[:DR], hs_dr, cnt['dr'], b_drdr)

    di2 = _lin_relu(di_data, W_lin_di, b_lin_di)
    dr2 = _lin_relu(dr_data, W_lin_dr, b_lin_dr)
    p1 = _lin_relu(p_data, W_lin_p, b_lin_p)
    g1 = _lin_relu(g_data, W_lin_g, b_lin_g)

    def layer2(nm, xa, xb, w, b):
        src, dst, nn, nd_pad = jobs[nm]
        c = cnt[nm]
        na = xa.shape[0]
        hs = jnp.concatenate([_mm_scale(xa, w, c[:na]), _mm_scale(xb, w, c[na:])])
        if hs.shape[0] < nd_pad:
            hs = jnp.pad(hs, ((0, nd_pad - hs.shape[0]), (0, 0)))
        agg = _gcn_seg_call(src, dst, hs, nd_pad)
        return _finish(agg[:nn], hs[:nn], c, b)

    h = layer2('didr', di2, dr2, W_drdi, b_drdi)
    di3, dr3 = h[:DI], h[DI:]
    p3 = layer2('dip', di2, p1, W_dip, b_dip)[DI:]
    p2 = layer2('drp', dr2, p1, W_drp, b_drp)[DR:]
    g3 = layer2('dig', di2, g1, W_dig, b_dig)[DI:]
    g2 = layer2('drg', dr2, g1, W_drg, b_drg)[DR:]

    xr = jnp.concatenate([di2, di1, di3, dr2, dr1, dr3, p3, p2, g3, g2])
    a = _rg_seg_call(rsrc_all, rdst_all, xr)
    return _combine(xr, a.reshape(18, NRG, 128), cnt_rg, W_root, W_rel, b_rgcn)
